# baseline (speedup 1.0000x reference)
import jax
import jax.numpy as jnp
from jax.experimental import pallas as pl
from jax.experimental.pallas import tpu as pltpu

_B, _F, _T = 128, 80, 4096
_FREQ_MASKS = 2
_TIME_MASKS = 10
_BB = 8
_G = 4
_NU = _F // _G
_NG = _B // _BB


def _spans(fs_ref, fl_ref):
    s0, l0 = fs_ref[0], fl_ref[0]
    s1, l1 = fs_ref[1], fl_ref[1]
    e0, e1 = s0 + l0, s1 + l1
    first = s0 <= s1
    a_s = jnp.where(first, s0, s1)
    a_e = jnp.where(first, e0, e1)
    b_s = jnp.where(first, s1, s0)
    b_e = jnp.where(first, e1, e0)
    merged = b_s <= a_e
    i0s = a_s
    i0e = jnp.where(merged, jnp.maximum(a_e, b_e), a_e)
    i1s = jnp.where(merged, _F, b_s)
    i1e = jnp.where(merged, _F, b_e)
    return ((jnp.int32(0), i0s), (i0e, i1s), (i1e, jnp.int32(_F)))


def _issue(spans, g_src, slot, x_ref, xbuf_ref, sem_ref, do_wait):
    for ss, se in spans:
        ustart = ss // _G
        uend = (se + _G - 1) // _G
        for k in range(_NU):
            u = ustart + k

            @pl.when(u < uend)
            def _(u=u):
                cp = pltpu.make_async_copy(
                    x_ref.at[pl.ds(g_src * _BB, _BB), pl.ds(u, 1), :, :],
                    xbuf_ref.at[slot, :, pl.ds(u, 1), :, :],
                    sem_ref.at[slot],
                )
                if do_wait:
                    cp.wait()
                else:
                    cp.start()


def _body(xlen_ref, fs_ref, fl_ref, ts_ref, tl_ref, x_ref, o_ref,
          xbuf_ref, tkeep_ref, sem_ref):
    g = pl.program_id(0)
    spans = _spans(fs_ref, fl_ref)

    @pl.when(g == 0)
    def _():
        _issue(spans, 0, 0, x_ref, xbuf_ref, sem_ref, do_wait=False)
        t_io = jax.lax.broadcasted_iota(jnp.int32, (1, _T), 1)
        tk = jnp.ones((1, _T), jnp.float32)
        for i in range(_TIME_MASKS):
            s = ts_ref[i]
            e = s + tl_ref[i]
            tk = jnp.where((t_io >= s) & (t_io < e), 0.0, tk)
        tkeep_ref[...] = tk

    @pl.when(g + 1 < _NG)
    def _():
        _issue(spans, g + 1, (g + 1) % 2, x_ref, xbuf_ref, sem_ref,
               do_wait=False)

    _issue(spans, g, g % 2, x_ref, xbuf_ref, sem_ref, do_wait=True)

    f_io = jax.lax.broadcasted_iota(jnp.int32, (_F, 1), 0)
    fkeep = jnp.ones((_F, 1), jnp.float32)
    for i in range(_FREQ_MASKS):
        s = fs_ref[i]
        e = s + fl_ref[i]
        fkeep = jnp.where((f_io >= s) & (f_io < e), 0.0, fkeep)

    t_io = jax.lax.broadcasted_iota(jnp.int32, (1, _T), 1)
    slot = g % 2
    for lb in range(_BB):
        xl = xlen_ref[g * _BB + lb]
        tkeep = jnp.where(t_io < xl, tkeep_ref[...], 1.0)
        keep = fkeep * tkeep
        xv = xbuf_ref[slot, lb].reshape(_F, _T)
        o_ref[lb] = jnp.where(keep != 0.0, xv, 0.0)


def kernel(x, x_len, freq_starts, freq_lengths, time_starts, time_lengths):
    x4 = x.reshape(_B, _NU, _G, _T)
    grid_spec = pltpu.PrefetchScalarGridSpec(
        num_scalar_prefetch=5,
        grid=(_NG,),
        in_specs=[pl.BlockSpec(memory_space=pl.ANY)],
        out_specs=pl.BlockSpec((_BB, _F, _T), lambda g, *_: (g, 0, 0)),
        scratch_shapes=[
            pltpu.VMEM((2, _BB, _NU, _G, _T), jnp.float32),
            pltpu.VMEM((1, _T), jnp.float32),
            pltpu.SemaphoreType.DMA((2,)),
        ],
    )
    return pl.pallas_call(
        _body,
        grid_spec=grid_spec,
        out_shape=jax.ShapeDtypeStruct((_B, _F, _T), jnp.float32),
    )(x_len, freq_starts, freq_lengths, time_starts, time_lengths, x4)


# speedup vs baseline: 1.1372x; 1.1372x over previous
import functools

import jax
import jax.numpy as jnp
from jax import lax
from jax.experimental import pallas as pl
from jax.experimental.pallas import tpu as pltpu
from jax.experimental.pallas import tpu_sc as plsc

_B, _F, _T = 128, 80, 4096
_ROWS = _B * _F
_NC, _NS, _L = 2, 16, 16
_NW = _NC * _NS
_RPW = _ROWS // _NW
_BPW = _RPW // _F
_SLAB = 8
_NSLAB = _F // _SLAB
_TIME_MASKS = 10


def _sc_body(x_hbm, xlen_hbm, par_hbm, out_hbm,
             par_v, xlen_v, buf_v, zbuf_v):
    wid = lax.axis_index("s") * _NC + lax.axis_index("c")

    pltpu.sync_copy(par_hbm, par_v)
    j16 = wid // 4
    pltpu.sync_copy(xlen_hbm.at[pl.ds(j16 * _L, _L)], xlen_v.at[pl.ds(0, _L)])
    p0 = par_v[pl.ds(0, _L)]
    tsv = par_v[pl.ds(_L, _L)]
    tlv = par_v[pl.ds(2 * _L, _L)]
    s0 = p0[0]
    s1 = p0[1]
    e0 = s0 + p0[2]
    e1 = s1 + p0[3]
    first = s0 <= s1
    a_s = jnp.where(first, s0, s1)
    a_e = jnp.where(first, e0, e1)
    b_s = jnp.where(first, s1, s0)
    b_e = jnp.where(first, e1, e0)
    merged = b_s <= a_e
    i0s = a_s
    i0e = jnp.where(merged, jnp.maximum(a_e, b_e), a_e)
    i1s = jnp.where(merged, _F, b_s)
    i1e = jnp.where(merged, _F, b_e)

    zeros16 = jnp.zeros((_L,), jnp.float32)

    def _zrow(r, _):
        def _zcol(j, _):
            zbuf_v[r, pl.ds(j * _L, _L)] = zeros16
            return 0
        return lax.fori_loop(0, _T // _L, _zcol, 0)
    lax.fori_loop(0, _SLAB, _zrow, 0)

    lane = lax.iota(jnp.int32, _L)

    for lb in range(_BPW):
        b_loc = (wid % 4) * 4 + lb
        xl = xlen_v[pl.ds(b_loc, _L)][0]
        clips = []
        for i in range(_TIME_MASKS):
            ce = jnp.minimum(tsv[i] + tlv[i], xl)
            cs = jnp.minimum(tsv[i], ce)
            clips.append((cs, ce))
        row0 = wid * _RPW + lb * _F

        def _slab(s, _, clips=clips, row0=row0):
            f0 = s * _SLAB
            rbase = row0 + f0
            full = ((f0 >= i0s) & (f0 + _SLAB <= i0e)) | (
                (f0 >= i1s) & (f0 + _SLAB <= i1e))

            @pl.when(full)
            def _():
                pltpu.sync_copy(zbuf_v, out_hbm.at[pl.ds(rbase, _SLAB)])

            @pl.when(jnp.logical_not(full))
            def _():
                pltpu.sync_copy(x_hbm.at[pl.ds(rbase, _SLAB)], buf_v)
                for r in range(_SLAB):
                    f = f0 + r
                    in_mask = ((f >= i0s) & (f < i0e)) | ((f >= i1s) &
                                                          (f < i1e))

                    @pl.when(in_mask)
                    def _(r=r):
                        def _zc(j, _):
                            buf_v[r, pl.ds(j * _L, _L)] = zeros16
                            return 0
                        lax.fori_loop(0, _T // _L, _zc, 0)

                for cs, ce in clips:
                    p0 = (cs // _L) * _L
                    nch = (ce + _L - 1) // _L - cs // _L

                    def _ch(k, _, cs=cs, ce=ce, p0=p0):
                        p = p0 + k * _L
                        col = p + lane
                        msk = (col >= cs) & (col < ce)
                        for r in range(_SLAB):
                            v = buf_v[r, pl.ds(p, _L)]
                            buf_v[r, pl.ds(p, _L)] = jnp.where(msk, 0.0, v)
                        return 0
                    lax.fori_loop(0, nch, _ch, 0)

                pltpu.sync_copy(buf_v, out_hbm.at[pl.ds(rbase, _SLAB)])
            return 0

        lax.fori_loop(0, _NSLAB, _slab, 0)


def kernel(x, x_len, freq_starts, freq_lengths, time_starts, time_lengths):
    x2 = x.reshape(_ROWS, _T)
    par = jnp.concatenate([
        freq_starts.astype(jnp.int32),
        freq_lengths.astype(jnp.int32),
        jnp.zeros((12,), jnp.int32),
        time_starts.astype(jnp.int32),
        jnp.zeros((6,), jnp.int32),
        time_lengths.astype(jnp.int32),
        jnp.zeros((6,), jnp.int32),
    ])
    mesh = plsc.VectorSubcoreMesh(core_axis_name="c", subcore_axis_name="s")
    run = functools.partial(
        pl.kernel,
        mesh=mesh,
        out_type=jax.ShapeDtypeStruct((_ROWS, _T), jnp.float32),
        scratch_types=[
            pltpu.VMEM((48,), jnp.int32),
            pltpu.VMEM((2 * _L,), jnp.int32),
            pltpu.VMEM((_SLAB, _T), jnp.float32),
            pltpu.VMEM((_SLAB, _T), jnp.float32),
        ],
    )(_sc_body)
    out2 = run(x2, x_len, par)
    return out2.reshape(_B, _F, _T)
